# Initial kernel scaffold; baseline (speedup 1.0000x reference)
import jax
import jax.numpy as jnp
from jax.experimental import pallas as pl


def kernel(token_ids, weight):
    raise NotImplementedError("write your pallas kernel here")


# speedup vs baseline: 1.1108x; 1.1108x over previous
import functools

import jax
import jax.numpy as jnp
from jax import lax
from jax.experimental import pallas as pl
from jax.experimental.pallas import tpu as pltpu
from jax.experimental.pallas import tpu_sc as plsc

NUM_TOKENS = 16384
SEQ = 50
DIM = 32
B = NUM_TOKENS * SEQ

_info = plsc.get_sparse_core_info()
NC, NS = _info.num_cores, _info.num_subcores
NW = NC * NS
B_PER_W = B // NW
CHUNK = 3200
NCHUNK = B_PER_W // CHUNK


def _emb_body(idx_hbm, table_hbm, out_hbm, idx_v, rows_v, sem):
    wid = lax.axis_index("s") * NC + lax.axis_index("c")
    base = wid * B_PER_W

    def body(ci, _):
        b0 = base + ci * CHUNK
        pltpu.sync_copy(idx_hbm.at[pl.ds(b0, CHUNK)], idx_v)
        pltpu.async_copy(table_hbm.at[idx_v], rows_v, sem).wait()
        pltpu.sync_copy(rows_v, out_hbm.at[pl.ds(b0, CHUNK)])
        return 0

    lax.fori_loop(0, NCHUNK, body, 0)


_emb = functools.partial(
    pl.kernel,
    mesh=plsc.VectorSubcoreMesh(core_axis_name="c", subcore_axis_name="s"),
    out_type=jax.ShapeDtypeStruct((B, DIM), jnp.float32),
    scratch_types=[
        pltpu.VMEM((CHUNK,), jnp.int32),
        pltpu.VMEM((CHUNK, DIM), jnp.float32),
        pltpu.SemaphoreType.DMA,
    ],
    compiler_params=pltpu.CompilerParams(use_tc_tiling_on_sc=False),
)(_emb_body)


@jax.jit
def kernel(token_ids, weight):
    idx = token_ids.astype(jnp.int32).reshape(B)
    out = _emb(idx, weight)
    return out.reshape(NUM_TOKENS, SEQ, DIM)


# speedup vs baseline: 1.1123x; 1.0014x over previous
import functools

import jax
import jax.numpy as jnp
from jax import lax
from jax.experimental import pallas as pl
from jax.experimental.pallas import tpu as pltpu
from jax.experimental.pallas import tpu_sc as plsc

NUM_TOKENS = 16384
SEQ = 50
DIM = 32
B = NUM_TOKENS * SEQ

_info = plsc.get_sparse_core_info()
NC, NS = _info.num_cores, _info.num_subcores
NW = NC * NS
B_PER_W = B // NW
CHUNK = 1600
NCHUNK = B_PER_W // CHUNK


def _emb_body(idx_hbm, table_hbm, out_hbm, idx_v, rows_v, sg0, sg1, ss0, ss1):
    wid = lax.axis_index("s") * NC + lax.axis_index("c")
    base = wid * B_PER_W

    sg = (sg0, sg1)
    ss = (ss0, ss1)

    pltpu.sync_copy(idx_hbm.at[pl.ds(base, B_PER_W)], idx_v)

    def start_gather(ci):
        s = ci % 2
        return pltpu.async_copy(
            table_hbm.at[idx_v.at[pl.ds(ci * CHUNK, CHUNK)]],
            rows_v.at[s],
            sg[s],
        )

    def start_store(ci):
        s = ci % 2
        return pltpu.async_copy(
            rows_v.at[s],
            out_hbm.at[pl.ds(base + ci * CHUNK, CHUNK)],
            ss[s],
        )

    gathers = [None] * NCHUNK
    stores = [None] * NCHUNK
    gathers[0] = start_gather(0)
    for ci in range(NCHUNK):
        if ci + 1 < NCHUNK:
            if ci >= 1:
                stores[ci - 1].wait()
            gathers[ci + 1] = start_gather(ci + 1)
        gathers[ci].wait()
        stores[ci] = start_store(ci)
    stores[NCHUNK - 2].wait()
    stores[NCHUNK - 1].wait()


_emb = functools.partial(
    pl.kernel,
    mesh=plsc.VectorSubcoreMesh(core_axis_name="c", subcore_axis_name="s"),
    out_type=jax.ShapeDtypeStruct((B, DIM), jnp.float32),
    scratch_types=[
        pltpu.VMEM((B_PER_W,), jnp.int32),
        pltpu.VMEM((2, CHUNK, DIM), jnp.float32),
        pltpu.SemaphoreType.DMA,
        pltpu.SemaphoreType.DMA,
        pltpu.SemaphoreType.DMA,
        pltpu.SemaphoreType.DMA,
    ],
    compiler_params=pltpu.CompilerParams(use_tc_tiling_on_sc=False),
)(_emb_body)


@jax.jit
def kernel(token_ids, weight):
    idx = token_ids.astype(jnp.int32).reshape(B)
    out = _emb(idx, weight)
    return out.reshape(NUM_TOKENS, SEQ, DIM)


# speedup vs baseline: 1.4988x; 1.3474x over previous
import functools

import jax
import jax.numpy as jnp
from jax import lax
from jax.experimental import pallas as pl
from jax.experimental.pallas import tpu as pltpu
from jax.experimental.pallas import tpu_sc as plsc

NUM_TOKENS = 16384
SEQ = 50
DIM = 32

_info = plsc.get_sparse_core_info()
NC, NS = _info.num_cores, _info.num_subcores
NW = NC * NS
TW = NUM_TOKENS // NW


def _emb_body(idxT_hbm, table_hbm, outT_hbm, idx_v, rows_v, out_v, sg0, sg1, ss0, ss1):
    wid = lax.axis_index("s") * NC + lax.axis_index("c")
    t0 = wid * TW
    sg = (sg0, sg1)
    ss = (ss0, ss1)
    iota16 = lax.iota(jnp.int32, 16)

    pltpu.sync_copy(idxT_hbm.at[:, pl.ds(t0, TW)], idx_v)

    def start_gather(s, slot):
        pltpu.async_copy(table_hbm.at[idx_v.at[s]], rows_v.at[slot], sg[slot])

    def wait_gather(slot):
        pltpu.make_async_copy(
            table_hbm.at[idx_v.at[0]], rows_v.at[slot], sg[slot]
        ).wait()

    def start_store(s, slot):
        pltpu.async_copy(out_v.at[slot], outT_hbm.at[s, :, pl.ds(t0, TW)], ss[slot])

    def wait_store(slot):
        pltpu.make_async_copy(
            out_v.at[slot], outT_hbm.at[0, :, pl.ds(t0, TW)], ss[slot]
        ).wait()

    def transpose(slot):
        def jbody(j, carry):
            row_idx = iota16 + j * 16
            for d in range(DIM):
                col_idx = jnp.full((16,), d, dtype=jnp.int32)
                v = plsc.load_gather(rows_v.at[slot], [row_idx, col_idx])
                out_v[slot, d, pl.ds(j * 16, 16)] = v
            return carry

        lax.fori_loop(0, TW // 16, jbody, 0)

    def do_iter(s, slot, wait_prev_store, next_gather):
        if next_gather:
            start_gather(s + 1, 1 - slot)
        wait_gather(slot)
        if wait_prev_store:
            wait_store(slot)
        transpose(slot)
        start_store(s, slot)

    start_gather(0, 0)
    do_iter(0, 0, False, True)
    do_iter(1, 1, False, True)

    def body(i, carry):
        do_iter(2 * i, 0, True, True)
        do_iter(2 * i + 1, 1, True, True)
        return carry

    lax.fori_loop(1, (SEQ - 2) // 2, body, 0)

    do_iter(SEQ - 2, 0, True, True)
    do_iter(SEQ - 1, 1, True, False)
    wait_store(0)
    wait_store(1)


_emb = functools.partial(
    pl.kernel,
    mesh=plsc.VectorSubcoreMesh(core_axis_name="c", subcore_axis_name="s"),
    out_type=jax.ShapeDtypeStruct((SEQ, DIM, NUM_TOKENS), jnp.float32),
    scratch_types=[
        pltpu.VMEM((SEQ, TW), jnp.int32),
        pltpu.VMEM((2, TW, DIM), jnp.float32),
        pltpu.VMEM((2, DIM, TW), jnp.float32),
        pltpu.SemaphoreType.DMA,
        pltpu.SemaphoreType.DMA,
        pltpu.SemaphoreType.DMA,
        pltpu.SemaphoreType.DMA,
    ],
    compiler_params=pltpu.CompilerParams(
        use_tc_tiling_on_sc=False, needs_layout_passes=False
    ),
)(_emb_body)


@jax.jit
def kernel(token_ids, weight):
    idxT = token_ids.astype(jnp.int32).T
    outT = _emb(idxT, weight)
    return jnp.transpose(outT, (2, 0, 1))


# speedup vs baseline: 1.6472x; 1.0990x over previous
import functools

import jax
import jax.numpy as jnp
from jax import lax
from jax.experimental import pallas as pl
from jax.experimental.pallas import tpu as pltpu
from jax.experimental.pallas import tpu_sc as plsc

NUM_TOKENS = 16384
SEQ = 50
DIM = 32
NUM_ROWS = 1000000

_info = plsc.get_sparse_core_info()
NC, NS = _info.num_cores, _info.num_subcores
NW = NC * NS
TW = NUM_TOKENS // NW
TT = TW // 128
DT = DIM // 8


def _emb_body(idxT_hbm, table_hbm, out_hbm, idx_v, rows_v, out_v, sg0, sg1, ss0, ss1):
    wid = lax.axis_index("s") * NC + lax.axis_index("c")
    t0 = wid * TW
    tt0 = wid * TT
    sg = (sg0, sg1)
    ss = (ss0, ss1)
    iota16 = lax.iota(jnp.int32, 16)

    pltpu.sync_copy(idxT_hbm.at[:, pl.ds(t0, TW)], idx_v)

    def start_gather(s, slot):
        pltpu.async_copy(table_hbm.at[idx_v.at[s]], rows_v.at[slot], sg[slot])

    def wait_gather(slot):
        pltpu.make_async_copy(
            table_hbm.at[idx_v.at[0]], rows_v.at[slot], sg[slot]
        ).wait()

    def start_store(s, slot):
        pltpu.async_copy(
            out_v.at[slot], out_hbm.at[s, :, pl.ds(tt0, TT), :, :], ss[slot]
        )

    def wait_store(slot):
        pltpu.make_async_copy(
            out_v.at[0], out_hbm.at[0, :, pl.ds(tt0, TT), :, :], ss[slot]
        ).wait()

    def transpose(slot):
        def jbody(j, carry):
            jo = j // 8
            ji = (j % 8) * 16
            row_idx = iota16 + j * 16
            for d in range(DIM):
                col_idx = jnp.full((16,), d, dtype=jnp.int32)
                v = plsc.load_gather(rows_v.at[slot], [row_idx, col_idx])
                out_v[slot, d // 8, jo, d % 8, pl.ds(ji, 16)] = v
            return carry

        lax.fori_loop(0, TW // 16, jbody, 0)

    def do_iter(s, slot, wait_prev_store, next_gather):
        if next_gather:
            start_gather(s + 1, 1 - slot)
        wait_gather(slot)
        if wait_prev_store:
            wait_store(slot)
        transpose(slot)
        start_store(s, slot)

    start_gather(0, 0)
    do_iter(0, 0, False, True)
    do_iter(1, 1, False, True)

    def body(i, carry):
        do_iter(2 * i, 0, True, True)
        do_iter(2 * i + 1, 1, True, True)
        return carry

    lax.fori_loop(1, (SEQ - 2) // 2, body, 0)

    do_iter(SEQ - 2, 0, True, True)
    do_iter(SEQ - 1, 1, True, False)
    wait_store(0)
    wait_store(1)


_emb = functools.partial(
    pl.kernel,
    mesh=plsc.VectorSubcoreMesh(core_axis_name="c", subcore_axis_name="s"),
    out_type=jax.ShapeDtypeStruct(
        (SEQ, DT, NUM_TOKENS // 128, 8, 128), jnp.float32
    ),
    scratch_types=[
        pltpu.VMEM((SEQ, TW), jnp.int32),
        pltpu.VMEM((2, TW, DIM), jnp.float32),
        pltpu.VMEM((2, DT, TT, 8, 128), jnp.float32),
        pltpu.SemaphoreType.DMA,
        pltpu.SemaphoreType.DMA,
        pltpu.SemaphoreType.DMA,
        pltpu.SemaphoreType.DMA,
    ],
    compiler_params=pltpu.CompilerParams(
        use_tc_tiling_on_sc=False, needs_layout_passes=False
    ),
)(_emb_body)


@jax.jit
def kernel(token_ids, weight):
    idxT = jnp.minimum(token_ids.astype(jnp.int32).T, jnp.int32(NUM_ROWS - 1))
    out5 = _emb(idxT, weight)
    out = jnp.transpose(out5, (2, 4, 0, 1, 3)).reshape(NUM_TOKENS, SEQ, DIM)
    return out


# speedup vs baseline: 2.1500x; 1.3053x over previous
import functools

import jax
import jax.numpy as jnp
from jax import lax
from jax.experimental import pallas as pl
from jax.experimental.pallas import tpu as pltpu
from jax.experimental.pallas import tpu_sc as plsc

NUM_TOKENS = 16384
SEQ = 50
DIM = 32
NUM_ROWS = 1000000

_info = plsc.get_sparse_core_info()
NC, NS = _info.num_cores, _info.num_subcores
NW = NC * NS
TW = NUM_TOKENS // NW
TT = TW // 128
DT = DIM // 8


def _emb_body(idx_hbm, table_hbm, out_hbm, idx_v, rows_v, out_v, si, sg0, sg1, ss0, ss1):
    wid = lax.axis_index("s") * NC + lax.axis_index("c")
    t0 = wid * TW
    tt0 = wid * TT
    sg = (sg0, sg1)
    ss = (ss0, ss1)
    iota16 = lax.iota(jnp.int32, 16)
    col_idx = [jnp.full((16,), d, dtype=jnp.int32) for d in range(DIM)]

    for s in range(SEQ):
        pltpu.async_copy(
            idx_hbm.at[pl.ds(s * NUM_TOKENS + t0, TW)], idx_v.at[s], si
        )
    for s in range(SEQ):
        pltpu.make_async_copy(
            idx_hbm.at[pl.ds(t0, TW)], idx_v.at[0], si
        ).wait()

    def start_gather(s, slot):
        pltpu.async_copy(table_hbm.at[idx_v.at[s]], rows_v.at[slot], sg[slot])

    def wait_gather(slot):
        pltpu.make_async_copy(
            table_hbm.at[idx_v.at[0]], rows_v.at[slot], sg[slot]
        ).wait()

    def start_store(s, slot):
        pltpu.async_copy(
            out_v.at[slot], out_hbm.at[s, :, pl.ds(tt0, TT), :, :], ss[slot]
        )

    def wait_store(slot):
        pltpu.make_async_copy(
            out_v.at[0], out_hbm.at[0, :, pl.ds(tt0, TT), :, :], ss[slot]
        ).wait()

    def transpose(slot):
        def jbody(j, carry):
            jo = j // 8
            ji = (j % 8) * 16
            row_idx = iota16 + j * 16
            vals = [
                plsc.load_gather(rows_v.at[slot], [row_idx, col_idx[d]])
                for d in range(DIM)
            ]
            for d in range(DIM):
                out_v[slot, d // 8, jo, d % 8, pl.ds(ji, 16)] = vals[d]
            return carry

        lax.fori_loop(0, TW // 16, jbody, 0)

    def do_iter(s, slot, wait_prev_store, next_gather):
        if next_gather:
            start_gather(s + 1, 1 - slot)
        wait_gather(slot)
        if wait_prev_store:
            wait_store(slot)
        transpose(slot)
        start_store(s, slot)

    start_gather(0, 0)
    do_iter(0, 0, False, True)
    do_iter(1, 1, False, True)

    def body(i, carry):
        do_iter(2 * i, 0, True, True)
        do_iter(2 * i + 1, 1, True, True)
        return carry

    lax.fori_loop(1, (SEQ - 2) // 2, body, 0)

    do_iter(SEQ - 2, 0, True, True)
    do_iter(SEQ - 1, 1, True, False)
    wait_store(0)
    wait_store(1)


_emb = functools.partial(
    pl.kernel,
    mesh=plsc.VectorSubcoreMesh(core_axis_name="c", subcore_axis_name="s"),
    out_type=jax.ShapeDtypeStruct(
        (SEQ, DT, NUM_TOKENS // 128, 8, 128), jnp.float32
    ),
    scratch_types=[
        pltpu.VMEM((SEQ, TW), jnp.int32),
        pltpu.VMEM((2, TW, DIM), jnp.float32),
        pltpu.VMEM((2, DT, TT, 8, 128), jnp.float32),
        pltpu.SemaphoreType.DMA,
        pltpu.SemaphoreType.DMA,
        pltpu.SemaphoreType.DMA,
        pltpu.SemaphoreType.DMA,
        pltpu.SemaphoreType.DMA,
    ],
    compiler_params=pltpu.CompilerParams(
        use_tc_tiling_on_sc=False, needs_layout_passes=False
    ),
)(_emb_body)


@jax.jit
def kernel(token_ids, weight):
    idx1 = jnp.minimum(
        token_ids.astype(jnp.int32).T, jnp.int32(NUM_ROWS - 1)
    ).reshape(-1)
    out5 = _emb(idx1, weight)
    out = jnp.transpose(out5, (2, 4, 0, 1, 3)).reshape(NUM_TOKENS, SEQ, DIM)
    return out
